# baseline (speedup 1.0000x reference)
import jax
import jax.numpy as jnp
from jax import lax
from jax.experimental import pallas as pl
from jax.experimental.pallas import tpu as pltpu
from jax.experimental.pallas import tpu_sc as plsc

F32 = jnp.float32
I32 = jnp.int32

_E, _K, _D, _H = 64, 2, 768, 128
_N = 8192
_NSLOT = _N * _K
_BLK = 128
_NB = _NSLOT // _BLK + _E
_PADN = _NB * _BLK
_TB = 512

_NC, _NSUB = 2, 16
_NW = _NC * _NSUB


def _shared_body(x_ref, swg_ref, swu_ref, swd_ref, sh_ref):
    xb = x_ref[...]
    g = jnp.dot(xb, swg_ref[...], preferred_element_type=F32)
    u = jnp.dot(xb, swu_ref[...], preferred_element_type=F32)
    h = g * jax.nn.sigmoid(g) * u
    sh_ref[...] = jnp.dot(h, swd_ref[...], preferred_element_type=F32)


def _shared_mm(xf, swg, swu, swd):
    nta = _N // _TB
    return pl.pallas_call(
        _shared_body,
        grid=(nta,),
        in_specs=[
            pl.BlockSpec((_TB, _D), lambda i: (i, 0)),
            pl.BlockSpec((_D, _H), lambda i: (0, 0)),
            pl.BlockSpec((_D, _H), lambda i: (0, 0)),
            pl.BlockSpec((_H, _D), lambda i: (0, 0)),
        ],
        out_specs=pl.BlockSpec((_TB, _D), lambda i: (i, 0)),
        out_shape=jax.ShapeDtypeStruct((_N, _D), F32),
    )(xf, swg, swu, swd)


def _router_body(x_ref, wr_ref, lv_ref,
                 e1_ref, e2_ref, w1_ref, w2_ref):
    xb = x_ref[...]
    logits = jnp.dot(xb, wr_ref[:_D, :], preferred_element_type=F32)
    logits = logits + jnp.dot(lv_ref[...], wr_ref[_D:, :],
                              preferred_element_type=F32)
    lane = lax.broadcasted_iota(I32, logits.shape, 1)
    m1 = jnp.max(logits, axis=1, keepdims=True)
    i1 = jnp.min(jnp.where(logits == m1, lane, _E), axis=1, keepdims=True)
    masked = jnp.where(lane == i1, -1e30, logits)
    m2 = jnp.max(masked, axis=1, keepdims=True)
    i2 = jnp.min(jnp.where(masked == m2, lane, _E), axis=1, keepdims=True)
    e1_ref[...] = i1
    e2_ref[...] = i2
    w1_ref[...] = jax.nn.sigmoid(m1)
    w2_ref[...] = jax.nn.sigmoid(m2)


def _router(xf, Wr, lv):
    nta = _N // _TB
    return pl.pallas_call(
        _router_body,
        grid=(nta,),
        in_specs=[
            pl.BlockSpec((_TB, _D), lambda i: (i, 0)),
            pl.BlockSpec((2 * _D, _E), lambda i: (0, 0)),
            pl.BlockSpec((1, _D), lambda i: (0, 0)),
        ],
        out_specs=[
            pl.BlockSpec((_TB, 1), lambda i: (i, 0)),
            pl.BlockSpec((_TB, 1), lambda i: (i, 0)),
            pl.BlockSpec((_TB, 1), lambda i: (i, 0)),
            pl.BlockSpec((_TB, 1), lambda i: (i, 0)),
        ],
        out_shape=[
            jax.ShapeDtypeStruct((_N, 1), I32),
            jax.ShapeDtypeStruct((_N, 1), I32),
            jax.ShapeDtypeStruct((_N, 1), F32),
            jax.ShapeDtypeStruct((_N, 1), F32),
        ],
    )(xf, Wr, lv)


def _meta_body(ef_ref, rank_ref, starts_ref, be_ref, counts_ref):
    i = pl.program_id(0)

    @pl.when(i == 0)
    def _():
        counts_ref[...] = jnp.zeros((_E, 128), F32)

    lt128 = (lax.broadcasted_iota(I32, (128, 128), 0) <
             lax.broadcasted_iota(I32, (128, 128), 1)).astype(F32)
    sub_iota = lax.broadcasted_iota(I32, (_E, 128), 0)
    counts = counts_ref[...]
    eb = ef_ref[...].reshape(8, 128)
    for r in range(8):
        er = eb[r:r + 1, :]
        oh = (sub_iota == er).astype(F32)
        rank_t = jnp.dot(oh, lt128, preferred_element_type=F32)
        dest_rank = jnp.sum(oh * (counts + rank_t), axis=0, keepdims=True)
        rank_ref[:, r:r + 1, :] = dest_rank.reshape(1, 1, 128)
        counts = counts + jnp.sum(oh, axis=1, keepdims=True)
    counts_ref[...] = counts

    @pl.when(i == pl.num_programs(0) - 1)
    def _():
        pc = jnp.ceil(counts / _BLK) * _BLK
        sl64 = (lax.broadcasted_iota(I32, (_E, _E), 1) <
                lax.broadcasted_iota(I32, (_E, _E), 0)).astype(F32)
        starts = jnp.dot(sl64, pc, preferred_element_type=F32)
        starts_ref[...] = starts
        sb = starts[:, 0:1] / _BLK
        jrow = lax.broadcasted_iota(I32, (_E, 256), 1).astype(F32)
        a = (jrow >= sb).astype(F32)
        be = jnp.sum(a, axis=0, keepdims=True) - 1.0
        be = jnp.clip(be, 0.0, float(_E - 1))
        be_ref[...] = jnp.broadcast_to(be, (8, 256)).astype(I32)


def _meta(ef3):
    nchunks = _NSLOT // 1024
    return pl.pallas_call(
        _meta_body,
        grid=(nchunks,),
        in_specs=[pl.BlockSpec((1, 8, 128), lambda i: (i, 0, 0))],
        out_specs=[
            pl.BlockSpec((1, 8, 128), lambda i: (i, 0, 0)),
            pl.BlockSpec((_E, 128), lambda i: (0, 0)),
            pl.BlockSpec((8, 256), lambda i: (0, 0)),
        ],
        out_shape=[
            jax.ShapeDtypeStruct((nchunks, 8, 128), F32),
            jax.ShapeDtypeStruct((_E, 128), F32),
            jax.ShapeDtypeStruct((8, 256), I32),
        ],
        scratch_shapes=[pltpu.VMEM((_E, 128), F32)],
    )(ef3)


def _dispatch_body(ef_h, rank_h, wf_h, starts_h,
                   wrow_h, dest_h,
                   ef_v, rank_v, wf_v, starts_v, dest_v, sem):
    wid = lax.axis_index("s") * _NC + lax.axis_index("c")
    loads = [
        pltpu.async_copy(ef_h.at[wid], ef_v, sem),
        pltpu.async_copy(rank_h.at[wid], rank_v, sem),
        pltpu.async_copy(wf_h.at[wid], wf_v, sem),
        pltpu.async_copy(starts_h, starts_v, sem),
    ]
    for d in loads:
        d.wait()
    for q in range(4):
        for t in range(8):
            sl = pl.ds(t * 16, 16)
            e16 = ef_v[q, sl]
            s16 = plsc.load_gather(starts_v, [e16])
            dest_v[q, sl] = (s16 + rank_v[q, sl]).astype(I32)
    stores = [pltpu.async_copy(dest_v, dest_h.at[wid], sem)]
    for q in range(4):
        stores.append(
            pltpu.async_copy(wf_v.at[q], wrow_h.at[dest_v.at[q]], sem))
    for d in stores:
        d.wait()


def _dispatch(ef3, rank3, wf3, starts):
    mesh = plsc.VectorSubcoreMesh(core_axis_name="c", subcore_axis_name="s", num_cores=_NC, num_subcores=_NSUB)
    return pl.kernel(
        _dispatch_body,
        out_type=[
            jax.ShapeDtypeStruct((_PADN,), F32),
            jax.ShapeDtypeStruct((_NW, 4, 128), I32),
        ],
        mesh=mesh,
        compiler_params=pltpu.CompilerParams(needs_layout_passes=False),
        scratch_types=[
            pltpu.VMEM((4, 128), I32),
            pltpu.VMEM((4, 128), F32),
            pltpu.VMEM((4, 128), F32),
            pltpu.VMEM((128,), F32),
            pltpu.VMEM((4, 128), I32),
            pltpu.SemaphoreType.DMA,
        ],
    )(ef3, rank3, wf3, starts)


_SROWS = _NSLOT // _NW
_SCH = 64
_SNCH = _SROWS // _SCH


def _scatter_rows_body(dest_h, xflat_h, xs_h, dest_v, rows_v, lsem, ssem):
    wid = lax.axis_index("s") * _NC + lax.axis_index("c")
    base = wid * _SROWS
    tok0 = base % _N
    pltpu.sync_copy(dest_h.at[wid], dest_v)

    def l_start(c):
        return pltpu.async_copy(
            xflat_h.at[pl.ds(tok0 + c * _SCH, _SCH)], rows_v.at[c % 2], lsem)

    def s_start(c):
        return pltpu.async_copy(
            rows_v.at[c % 2], xs_h.at[dest_v.at[c]], ssem)

    l_d = [None] * _SNCH
    s_d = [None] * _SNCH
    l_d[0] = l_start(0)
    for c in range(_SNCH):
        l_d[c].wait()
        if c + 1 < _SNCH:
            if c >= 1:
                s_d[c - 1].wait()
            l_d[c + 1] = l_start(c + 1)
        s_d[c] = s_start(c)
    s_d[_SNCH - 2].wait()
    s_d[_SNCH - 1].wait()


def _scatter_rows(dest3, xf):
    mesh = plsc.VectorSubcoreMesh(core_axis_name="c", subcore_axis_name="s", num_cores=_NC, num_subcores=_NSUB)
    return pl.kernel(
        _scatter_rows_body,
        out_type=jax.ShapeDtypeStruct((_PADN, _D), F32),
        mesh=mesh,
        compiler_params=pltpu.CompilerParams(needs_layout_passes=False),
        scratch_types=[
            pltpu.VMEM((_SNCH, _SCH), I32),
            pltpu.VMEM((2, _SCH, _D), F32),
            pltpu.SemaphoreType.DMA,
            pltpu.SemaphoreType.DMA,
        ],
    )(dest3, xf)


def _expert_body(be_ref, xs_ref, wg_ref, wu_ref, wd_ref, w_ref, ybuf_ref):
    xb = xs_ref[...].astype(jnp.bfloat16)
    g = jnp.dot(xb, wg_ref[0], preferred_element_type=F32)
    u = jnp.dot(xb, wu_ref[0], preferred_element_type=F32)
    h = g * jax.nn.sigmoid(g) * u
    y = jnp.dot(h.astype(jnp.bfloat16), wd_ref[0], preferred_element_type=F32)
    ybuf_ref[...] = y * w_ref[...]


def _expert_mm(be, xs, Wg, Wu, Wd, wrow_col):
    grid_spec = pltpu.PrefetchScalarGridSpec(
        num_scalar_prefetch=1,
        grid=(_NB,),
        in_specs=[
            pl.BlockSpec((_BLK, _D), lambda i, be: (i, 0)),
            pl.BlockSpec((1, _D, _H), lambda i, be: (be[i], 0, 0)),
            pl.BlockSpec((1, _D, _H), lambda i, be: (be[i], 0, 0)),
            pl.BlockSpec((1, _H, _D), lambda i, be: (be[i], 0, 0)),
            pl.BlockSpec((_BLK, 1), lambda i, be: (i, 0)),
        ],
        out_specs=pl.BlockSpec((_BLK, _D), lambda i, be: (i, 0)),
    )
    return pl.pallas_call(
        _expert_body,
        grid_spec=grid_spec,
        out_shape=jax.ShapeDtypeStruct((_PADN, _D), F32),
    )(be, xs, Wg, Wu, Wd, wrow_col)


_CH = 16
_CROWS = _N // _NW
_CNCH = _CROWS // _CH


def _combine_body(dint_h, sh_h, ybuf_h, out_h,
                  d_v, sh_v, g_v, ob_v, lsem, gsem, osem):
    wid = lax.axis_index("s") * _NC + lax.axis_index("c")
    base = wid * _CROWS
    pltpu.sync_copy(dint_h.at[pl.ds(wid * 2 * _CROWS, 2 * _CROWS)], d_v)

    def s_start(c):
        return pltpu.async_copy(
            sh_h.at[pl.ds(base + c * _CH, _CH)], sh_v.at[c % 2], lsem)

    def g_start(c):
        return pltpu.async_copy(
            ybuf_h.at[d_v.at[pl.ds(c * 2 * _CH, 2 * _CH)]], g_v.at[c % 2], gsem)

    def o_start(c):
        return pltpu.async_copy(
            ob_v.at[c % 2], out_h.at[pl.ds(base + c * _CH, _CH)], osem)

    s_d = [None] * _CNCH
    g_d = [None] * _CNCH
    o_d = [None] * _CNCH
    s_d[0] = s_start(0)
    g_d[0] = g_start(0)
    for c in range(_CNCH):
        s_d[c].wait()
        g_d[c].wait()
        if c + 1 < _CNCH:
            if c >= 1:
                o_d[c - 1].wait()
            s_d[c + 1] = s_start(c + 1)
            g_d[c + 1] = g_start(c + 1)
        b = c % 2

        def add_row(r, _):
            for t in range(_D // 16):
                sl = pl.ds(t * 16, 16)
                ob_v[b, r, sl] = (sh_v[b, r, sl] + g_v[b, 2 * r, sl]
                                  + g_v[b, 2 * r + 1, sl])
            return 0

        lax.fori_loop(0, _CH, add_row, 0)
        o_d[c] = o_start(c)
    o_d[_CNCH - 2].wait()
    o_d[_CNCH - 1].wait()


def _combine(dint, shared, ybuf):
    mesh = plsc.VectorSubcoreMesh(core_axis_name="c", subcore_axis_name="s", num_cores=_NC, num_subcores=_NSUB)
    return pl.kernel(
        _combine_body,
        out_type=jax.ShapeDtypeStruct((_N, _D), F32),
        mesh=mesh,
        compiler_params=pltpu.CompilerParams(needs_layout_passes=False),
        scratch_types=[
            pltpu.VMEM((2 * _CROWS,), I32),
            pltpu.VMEM((2, _CH, _D), F32),
            pltpu.VMEM((2, 2 * _CH, _D), F32),
            pltpu.VMEM((2, _CH, _D), F32),
            pltpu.SemaphoreType.DMA,
            pltpu.SemaphoreType.DMA,
            pltpu.SemaphoreType.DMA,
        ],
    )(dint, shared, ybuf)


def kernel(x, loop_idx, shared_wg, shared_wu, shared_wd, Wg, Wu, Wd,
           loop_table, Wr):
    Bx, Sx, Dx = x.shape
    xf = x.reshape(_N, _D)
    lv = lax.dynamic_slice_in_dim(loop_table, loop_idx, 1, axis=0)

    shared = _shared_mm(xf, shared_wg, shared_wu, shared_wd)
    e1, e2, w1, w2 = _router(xf, Wr, lv)
    ef = jnp.concatenate([e1[:, 0], e2[:, 0]])
    wf = jnp.concatenate([w1[:, 0], w2[:, 0]])

    rank3, starts_bc, be_bc = _meta(ef.reshape(_NSLOT // 1024, 8, 128))
    starts = jnp.pad(starts_bc[:, 0], (0, 128 - _E))
    be = be_bc[0, :_NB]

    wrow, dest3 = _dispatch(
        ef.reshape(_NW, 4, 128),
        rank3.reshape(_NW, 4, 128),
        wf.reshape(_NW, 4, 128),
        starts,
    )
    dest = dest3.reshape(_NSLOT)

    xs = _scatter_rows(dest3.reshape(_NW, _SNCH, _SCH), xf)
    ybuf = _expert_mm(be, xs, Wg.astype(jnp.bfloat16), Wu.astype(jnp.bfloat16),
                      Wd.astype(jnp.bfloat16), wrow.reshape(_PADN, 1))
    dint = jnp.stack([dest[:_N], dest[_N:]], axis=1).reshape(_NSLOT)
    outf = _combine(dint, shared, ybuf)
    return outf.reshape(Bx, Sx, Dx)


# speedup vs baseline: 1.0786x; 1.0786x over previous
import jax
import jax.numpy as jnp
from jax import lax
from jax.experimental import pallas as pl
from jax.experimental.pallas import tpu as pltpu
from jax.experimental.pallas import tpu_sc as plsc

F32 = jnp.float32
I32 = jnp.int32

_E, _K, _D, _H = 64, 2, 768, 128
_N = 8192
_NSLOT = _N * _K
_BLK = 128
_NB = _NSLOT // _BLK + _E
_PADN = _NB * _BLK
_TB = 512

_NC, _NSUB = 2, 16
_NW = _NC * _NSUB


def _shared_body(x_ref, swg_ref, swu_ref, swd_ref, sh_ref):
    xb = x_ref[...]
    g = jnp.dot(xb, swg_ref[...], preferred_element_type=F32)
    u = jnp.dot(xb, swu_ref[...], preferred_element_type=F32)
    h = g * jax.nn.sigmoid(g) * u
    sh_ref[...] = jnp.dot(h, swd_ref[...], preferred_element_type=F32)


def _shared_mm(xf, swg, swu, swd):
    nta = _N // _TB
    return pl.pallas_call(
        _shared_body,
        grid=(nta,),
        in_specs=[
            pl.BlockSpec((_TB, _D), lambda i: (i, 0)),
            pl.BlockSpec((_D, _H), lambda i: (0, 0)),
            pl.BlockSpec((_D, _H), lambda i: (0, 0)),
            pl.BlockSpec((_H, _D), lambda i: (0, 0)),
        ],
        out_specs=pl.BlockSpec((_TB, _D), lambda i: (i, 0)),
        out_shape=jax.ShapeDtypeStruct((_N, _D), F32),
    )(xf, swg, swu, swd)


def _router_body(x_ref, wr_ref, lv_ref,
                 e1_ref, e2_ref, w1_ref, w2_ref):
    xb = x_ref[...]
    logits = jnp.dot(xb, wr_ref[:_D, :], preferred_element_type=F32)
    logits = logits + jnp.dot(lv_ref[...], wr_ref[_D:, :],
                              preferred_element_type=F32)
    lane = lax.broadcasted_iota(I32, logits.shape, 1)
    m1 = jnp.max(logits, axis=1, keepdims=True)
    i1 = jnp.min(jnp.where(logits == m1, lane, _E), axis=1, keepdims=True)
    masked = jnp.where(lane == i1, -1e30, logits)
    m2 = jnp.max(masked, axis=1, keepdims=True)
    i2 = jnp.min(jnp.where(masked == m2, lane, _E), axis=1, keepdims=True)
    e1_ref[...] = i1
    e2_ref[...] = i2
    w1_ref[...] = jax.nn.sigmoid(m1)
    w2_ref[...] = jax.nn.sigmoid(m2)


def _router(xf, Wr, lv):
    nta = _N // _TB
    return pl.pallas_call(
        _router_body,
        grid=(nta,),
        in_specs=[
            pl.BlockSpec((_TB, _D), lambda i: (i, 0)),
            pl.BlockSpec((2 * _D, _E), lambda i: (0, 0)),
            pl.BlockSpec((1, _D), lambda i: (0, 0)),
        ],
        out_specs=[
            pl.BlockSpec((_TB, 1), lambda i: (i, 0)),
            pl.BlockSpec((_TB, 1), lambda i: (i, 0)),
            pl.BlockSpec((_TB, 1), lambda i: (i, 0)),
            pl.BlockSpec((_TB, 1), lambda i: (i, 0)),
        ],
        out_shape=[
            jax.ShapeDtypeStruct((_N, 1), I32),
            jax.ShapeDtypeStruct((_N, 1), I32),
            jax.ShapeDtypeStruct((_N, 1), F32),
            jax.ShapeDtypeStruct((_N, 1), F32),
        ],
    )(xf, Wr, lv)


def _meta_body(ef_ref, rank_ref, starts_ref, be_ref, counts_ref):
    i = pl.program_id(0)

    @pl.when(i == 0)
    def _():
        counts_ref[...] = jnp.zeros((_E, 128), F32)

    lt128 = (lax.broadcasted_iota(I32, (128, 128), 0) <
             lax.broadcasted_iota(I32, (128, 128), 1)).astype(F32)
    sub_iota = lax.broadcasted_iota(I32, (_E, 128), 0)
    counts = counts_ref[...]
    eb = ef_ref[...].reshape(8, 128)
    for r in range(8):
        er = eb[r:r + 1, :]
        oh = (sub_iota == er).astype(F32)
        rank_t = jnp.dot(oh, lt128, preferred_element_type=F32)
        dest_rank = jnp.sum(oh * (counts + rank_t), axis=0, keepdims=True)
        rank_ref[:, r:r + 1, :] = dest_rank.reshape(1, 1, 128)
        counts = counts + jnp.sum(oh, axis=1, keepdims=True)
    counts_ref[...] = counts

    @pl.when(i == pl.num_programs(0) - 1)
    def _():
        pc = jnp.ceil(counts / _BLK) * _BLK
        sl64 = (lax.broadcasted_iota(I32, (_E, _E), 1) <
                lax.broadcasted_iota(I32, (_E, _E), 0)).astype(F32)
        starts = jnp.dot(sl64, pc, preferred_element_type=F32)
        starts_ref[...] = starts
        sb = starts[:, 0:1] / _BLK
        jrow = lax.broadcasted_iota(I32, (_E, 256), 1).astype(F32)
        a = (jrow >= sb).astype(F32)
        be = jnp.sum(a, axis=0, keepdims=True) - 1.0
        be = jnp.clip(be, 0.0, float(_E - 1))
        be_ref[...] = jnp.broadcast_to(be, (8, 256)).astype(I32)


def _meta(ef3):
    nchunks = _NSLOT // 1024
    return pl.pallas_call(
        _meta_body,
        grid=(nchunks,),
        in_specs=[pl.BlockSpec((1, 8, 128), lambda i: (i, 0, 0))],
        out_specs=[
            pl.BlockSpec((1, 8, 128), lambda i: (i, 0, 0)),
            pl.BlockSpec((_E, 128), lambda i: (0, 0)),
            pl.BlockSpec((8, 256), lambda i: (0, 0)),
        ],
        out_shape=[
            jax.ShapeDtypeStruct((nchunks, 8, 128), F32),
            jax.ShapeDtypeStruct((_E, 128), F32),
            jax.ShapeDtypeStruct((8, 256), I32),
        ],
        scratch_shapes=[pltpu.VMEM((_E, 128), F32)],
    )(ef3)


def _dispatch_body(ef_h, rank_h, wf_h, starts_h,
                   wrow_h, dest_h,
                   ef_v, rank_v, wf_v, starts_v, dest_v, sem):
    wid = lax.axis_index("s") * _NC + lax.axis_index("c")
    loads = [
        pltpu.async_copy(ef_h.at[wid], ef_v, sem),
        pltpu.async_copy(rank_h.at[wid], rank_v, sem),
        pltpu.async_copy(wf_h.at[wid], wf_v, sem),
        pltpu.async_copy(starts_h, starts_v, sem),
    ]
    for d in loads:
        d.wait()
    for q in range(4):
        for t in range(8):
            sl = pl.ds(t * 16, 16)
            e16 = ef_v[q, sl]
            s16 = plsc.load_gather(starts_v, [e16])
            dest_v[q, sl] = (s16 + rank_v[q, sl]).astype(I32)
    stores = [pltpu.async_copy(dest_v, dest_h.at[wid], sem)]
    for q in range(4):
        stores.append(
            pltpu.async_copy(wf_v.at[q], wrow_h.at[dest_v.at[q]], sem))
    for d in stores:
        d.wait()


def _dispatch(ef3, rank3, wf3, starts):
    mesh = plsc.VectorSubcoreMesh(core_axis_name="c", subcore_axis_name="s", num_cores=_NC, num_subcores=_NSUB)
    return pl.kernel(
        _dispatch_body,
        out_type=[
            jax.ShapeDtypeStruct((_PADN,), F32),
            jax.ShapeDtypeStruct((_NW, 4, 128), I32),
        ],
        mesh=mesh,
        compiler_params=pltpu.CompilerParams(needs_layout_passes=False),
        scratch_types=[
            pltpu.VMEM((4, 128), I32),
            pltpu.VMEM((4, 128), F32),
            pltpu.VMEM((4, 128), F32),
            pltpu.VMEM((128,), F32),
            pltpu.VMEM((4, 128), I32),
            pltpu.SemaphoreType.DMA,
        ],
    )(ef3, rank3, wf3, starts)


_SROWS = _NSLOT // _NW
_SCH = 64
_SNCH = _SROWS // _SCH


def _scatter_rows_body(dest_h, xflat_h, xs_h, dest_v, rows_v, lsem, ssem):
    wid = lax.axis_index("s") * _NC + lax.axis_index("c")
    base = wid * _SROWS
    tok0 = base % _N
    pltpu.sync_copy(dest_h.at[wid], dest_v)

    def l_start(c):
        return pltpu.async_copy(
            xflat_h.at[pl.ds(tok0 + c * _SCH, _SCH)], rows_v.at[c % 2], lsem)

    def s_start(c):
        return pltpu.async_copy(
            rows_v.at[c % 2], xs_h.at[dest_v.at[c]], ssem)

    l_d = [None] * _SNCH
    s_d = [None] * _SNCH
    l_d[0] = l_start(0)
    for c in range(_SNCH):
        l_d[c].wait()
        if c + 1 < _SNCH:
            if c >= 1:
                s_d[c - 1].wait()
            l_d[c + 1] = l_start(c + 1)
        s_d[c] = s_start(c)
    s_d[_SNCH - 2].wait()
    s_d[_SNCH - 1].wait()


def _scatter_rows(dest3, xf):
    mesh = plsc.VectorSubcoreMesh(core_axis_name="c", subcore_axis_name="s", num_cores=_NC, num_subcores=_NSUB)
    return pl.kernel(
        _scatter_rows_body,
        out_type=jax.ShapeDtypeStruct((_PADN, _D), F32),
        mesh=mesh,
        compiler_params=pltpu.CompilerParams(needs_layout_passes=False),
        scratch_types=[
            pltpu.VMEM((_SNCH, _SCH), I32),
            pltpu.VMEM((2, _SCH, _D), F32),
            pltpu.SemaphoreType.DMA,
            pltpu.SemaphoreType.DMA,
        ],
    )(dest3, xf)


def _expert_body(be_ref, xs_ref, wg_ref, wu_ref, wd_ref, w_ref, ybuf_ref):
    xb = xs_ref[...].astype(jnp.bfloat16)
    g = jnp.dot(xb, wg_ref[0].astype(jnp.bfloat16), preferred_element_type=F32)
    u = jnp.dot(xb, wu_ref[0].astype(jnp.bfloat16), preferred_element_type=F32)
    h = g * jax.nn.sigmoid(g) * u
    y = jnp.dot(h.astype(jnp.bfloat16), wd_ref[0].astype(jnp.bfloat16),
                preferred_element_type=F32)
    ybuf_ref[...] = y * w_ref[...]


def _expert_mm(be, xs, Wg, Wu, Wd, wrow_col):
    grid_spec = pltpu.PrefetchScalarGridSpec(
        num_scalar_prefetch=1,
        grid=(_NB,),
        in_specs=[
            pl.BlockSpec((_BLK, _D), lambda i, be: (i, 0)),
            pl.BlockSpec((1, _D, _H), lambda i, be: (be[i], 0, 0)),
            pl.BlockSpec((1, _D, _H), lambda i, be: (be[i], 0, 0)),
            pl.BlockSpec((1, _H, _D), lambda i, be: (be[i], 0, 0)),
            pl.BlockSpec((_BLK, 1), lambda i, be: (i, 0)),
        ],
        out_specs=pl.BlockSpec((_BLK, _D), lambda i, be: (i, 0)),
    )
    return pl.pallas_call(
        _expert_body,
        grid_spec=grid_spec,
        out_shape=jax.ShapeDtypeStruct((_PADN, _D), F32),
    )(be, xs, Wg, Wu, Wd, wrow_col)


_CH = 16
_CROWS = _N // _NW
_CNCH = _CROWS // _CH


def _combine_body(dint_h, sh_h, ybuf_h, out_h,
                  d_v, sh_v, g_v, ob_v, lsem, gsem, osem):
    wid = lax.axis_index("s") * _NC + lax.axis_index("c")
    base = wid * _CROWS
    pltpu.sync_copy(dint_h.at[pl.ds(wid * 2 * _CROWS, 2 * _CROWS)], d_v)

    def s_start(c):
        return pltpu.async_copy(
            sh_h.at[pl.ds(base + c * _CH, _CH)], sh_v.at[c % 2], lsem)

    def g_start(c):
        return pltpu.async_copy(
            ybuf_h.at[d_v.at[pl.ds(c * 2 * _CH, 2 * _CH)]], g_v.at[c % 3], gsem)

    def o_start(c):
        return pltpu.async_copy(
            ob_v.at[c % 2], out_h.at[pl.ds(base + c * _CH, _CH)], osem)

    s_d = [None] * _CNCH
    g_d = [None] * _CNCH
    o_d = [None] * _CNCH
    s_d[0] = s_start(0)
    g_d[0] = g_start(0)
    g_d[1] = g_start(1)
    for c in range(_CNCH):
        s_d[c].wait()
        g_d[c].wait()
        if c + 2 < _CNCH:
            g_d[c + 2] = g_start(c + 2)
        if c + 1 < _CNCH:
            s_d[c + 1] = s_start(c + 1)
        if c >= 2:
            o_d[c - 2].wait()
        b = c % 2
        gb = c % 3

        def add_row(r, _):
            for t in range(_D // 16):
                sl = pl.ds(t * 16, 16)
                ob_v[b, r, sl] = (sh_v[b, r, sl] + g_v[gb, 2 * r, sl]
                                  + g_v[gb, 2 * r + 1, sl])
            return 0

        lax.fori_loop(0, _CH, add_row, 0)
        o_d[c] = o_start(c)
    o_d[_CNCH - 2].wait()
    o_d[_CNCH - 1].wait()


def _combine(dint, shared, ybuf):
    mesh = plsc.VectorSubcoreMesh(core_axis_name="c", subcore_axis_name="s", num_cores=_NC, num_subcores=_NSUB)
    return pl.kernel(
        _combine_body,
        out_type=jax.ShapeDtypeStruct((_N, _D), F32),
        mesh=mesh,
        compiler_params=pltpu.CompilerParams(needs_layout_passes=False),
        scratch_types=[
            pltpu.VMEM((2 * _CROWS,), I32),
            pltpu.VMEM((2, _CH, _D), F32),
            pltpu.VMEM((3, 2 * _CH, _D), F32),
            pltpu.VMEM((2, _CH, _D), F32),
            pltpu.SemaphoreType.DMA,
            pltpu.SemaphoreType.DMA,
            pltpu.SemaphoreType.DMA,
        ],
    )(dint, shared, ybuf)


def kernel(x, loop_idx, shared_wg, shared_wu, shared_wd, Wg, Wu, Wd,
           loop_table, Wr):
    Bx, Sx, Dx = x.shape
    xf = x.reshape(_N, _D)
    lv = lax.dynamic_slice_in_dim(loop_table, loop_idx, 1, axis=0)

    shared = _shared_mm(xf, shared_wg, shared_wu, shared_wd)
    e1, e2, w1, w2 = _router(xf, Wr, lv)
    ef = jnp.concatenate([e1[:, 0], e2[:, 0]])
    wf = jnp.concatenate([w1[:, 0], w2[:, 0]])

    rank3, starts_bc, be_bc = _meta(ef.reshape(_NSLOT // 1024, 8, 128))
    starts = jnp.pad(starts_bc[:, 0], (0, 128 - _E))
    be = be_bc[0, :_NB]

    wrow, dest3 = _dispatch(
        ef.reshape(_NW, 4, 128),
        rank3.reshape(_NW, 4, 128),
        wf.reshape(_NW, 4, 128),
        starts,
    )
    dest = dest3.reshape(_NSLOT)

    xs = _scatter_rows(dest3.reshape(_NW, _SNCH, _SCH), xf)
    ybuf = _expert_mm(be, xs, Wg, Wu, Wd, wrow.reshape(_PADN, 1))
    dint = jnp.stack([dest[:_N], dest[_N:]], axis=1).reshape(_NSLOT)
    outf = _combine(dint, shared, ybuf)
    return outf.reshape(Bx, Sx, Dx)
